# Initial kernel scaffold; baseline (speedup 1.0000x reference)
import jax
import jax.numpy as jnp
from jax.experimental import pallas as pl


def kernel(x, edge_index_sim, edge_index_disim, h1, h2, h3, h4, W1p, b1p, W1n, b1n, W2p, b2p, W2n, b2n, W3p, b3p, W3n, b3n, W4p, b4p, W4n, b4n, mu):
    raise NotImplementedError("write your pallas kernel here")


# speedup vs baseline: 13.3924x; 13.3924x over previous
import functools

import jax
import jax.numpy as jnp
from jax import lax
from jax.experimental import pallas as pl
from jax.experimental.pallas import tpu as pltpu
from jax.experimental.pallas import tpu_sc as plsc

N = 10000
E = 320000
EP = E + N
IN_DIMS = [3, 32, 64, 128, 256]
N_CLUSTERS = 30

NC, NS = 2, 16
NW = NC * NS
C = 128
NCHUNK = 81
EPT = NCHUNK * C
EPAD = EPT * NW
NPAD = 10240
ROWS_PT = NPAD // NS

_mesh = plsc.VectorSubcoreMesh(
    core_axis_name="c", subcore_axis_name="s", num_cores=NC, num_subcores=NS)
_sc_params = pltpu.CompilerParams(use_tc_tiling_on_sc=False)


def _make_seg2(d):
    @functools.partial(
        pl.kernel,
        out_type=jax.ShapeDtypeStruct((NC, 2, NPAD, d), jnp.float32),
        mesh=_mesh,
        compiler_params=_sc_params,
        scratch_types=[
            pltpu.VMEM((NCHUNK, C), jnp.int32),
            pltpu.VMEM((NCHUNK, C), jnp.int32),
            pltpu.VMEM((C, d), jnp.float32),
            pltpu.VMEM_SHARED((NPAD, d), jnp.float32),
            pltpu.SemaphoreType.DMA,
        ],
    )
    def k(t1, t2, src_h, dst_h, zc, out, idxs, idxd, rows, acc, sem):
        core = lax.axis_index("c")
        sub = lax.axis_index("s")
        pltpu.sync_copy(src_h.at[core, sub], idxs)
        pltpu.sync_copy(dst_h.at[core, sub], idxd)
        for p, tp in ((0, t1), (1, t2)):
            pltpu.sync_copy(zc, acc.at[pl.ds(sub * ROWS_PT, ROWS_PT)])
            plsc.subcore_barrier()

            @pl.loop(0, NCHUNK)
            def _(j):
                pltpu.async_copy(tp.at[idxs.at[j]], rows, sem).wait()
                pltpu.sync_copy(rows, acc.at[idxd.at[j]], add=True)

            plsc.subcore_barrier()
            pltpu.sync_copy(acc.at[pl.ds(sub * ROWS_PT, ROWS_PT)],
                            out.at[core, p, pl.ds(sub * ROWS_PT, ROWS_PT)])
        plsc.subcore_barrier()

    return k


def _make_seg1():
    d = 16

    @functools.partial(
        pl.kernel,
        out_type=jax.ShapeDtypeStruct((NC, 2, NPAD, d), jnp.float32),
        mesh=_mesh,
        compiler_params=_sc_params,
        scratch_types=[
            pltpu.VMEM((NCHUNK, C), jnp.int32),
            pltpu.VMEM((NCHUNK, C), jnp.int32),
            pltpu.VMEM((C, d), jnp.float32),
            pltpu.VMEM((C, d), jnp.float32),
            pltpu.VMEM_SHARED((NPAD, d), jnp.float32),
            pltpu.SemaphoreType.DMA,
        ],
    )
    def k(t1, src_h, dst_h, zc, ones_h, out, idxs, idxd, rows, ones_v, acc, sem):
        core = lax.axis_index("c")
        sub = lax.axis_index("s")
        pltpu.sync_copy(src_h.at[core, sub], idxs)
        pltpu.sync_copy(dst_h.at[core, sub], idxd)
        pltpu.sync_copy(ones_h, ones_v)
        for p in (0, 1):
            pltpu.sync_copy(zc, acc.at[pl.ds(sub * ROWS_PT, ROWS_PT)])
            plsc.subcore_barrier()

            if p == 0:
                @pl.loop(0, NCHUNK)
                def _(j):
                    pltpu.async_copy(t1.at[idxs.at[j]], rows, sem).wait()
                    pltpu.sync_copy(rows, acc.at[idxd.at[j]], add=True)
            else:
                @pl.loop(0, NCHUNK)
                def _(j):
                    pltpu.sync_copy(ones_v, acc.at[idxd.at[j]], add=True)

            plsc.subcore_barrier()
            pltpu.sync_copy(acc.at[pl.ds(sub * ROWS_PT, ROWS_PT)],
                            out.at[core, p, pl.ds(sub * ROWS_PT, ROWS_PT)])
        plsc.subcore_barrier()

    return k


_TCB = 2000


def _l1_body(sp, sn, cp, cn, x, wp, bp, wn, bn, h, o1, o2):
    rcp = 1.0 / jnp.maximum(cp[...], 1.0)
    rcn = 1.0 / jnp.maximum(cn[...], 1.0)
    agg_p = sp[...][:, :3] * rcp
    agg_n = sn[...][:, :3] * rcn
    xv = x[...]
    wpv = wp[...]
    wnv = wn[...]
    out_p = (jnp.dot(agg_p, wpv[:3], preferred_element_type=jnp.float32)
             + jnp.dot(xv, wpv[3:], preferred_element_type=jnp.float32)
             + bp[...])
    out_n = (jnp.dot(agg_n, wnv[:3], preferred_element_type=jnp.float32)
             + jnp.dot(xv, wnv[3:], preferred_element_type=jnp.float32)
             + bn[...])
    hv = h[...]
    o1[...] = (jnp.maximum(out_p, 0.0) + hv) * 0.5
    o2[...] = (jnp.maximum(out_n, 0.0) + hv) * 0.5


def _mid_body(d, sp1, sp2, sn1, sn2, cp, cn, x1, x2, wp, bp, wn, bn, h, o1, o2):
    rcp = 1.0 / jnp.maximum(cp[...], 1.0)
    rcn = 1.0 / jnp.maximum(cn[...], 1.0)
    ap1 = sp1[...] * rcp
    ap2 = sp2[...] * rcp
    an1 = sn1[...] * rcn
    an2 = sn2[...] * rcn
    wpv = wp[...]
    wnv = wn[...]
    out_p = (jnp.dot(ap1, wpv[:d], preferred_element_type=jnp.float32)
             + jnp.dot(an2, wpv[d:2 * d], preferred_element_type=jnp.float32)
             + jnp.dot(x1[...], wpv[2 * d:], preferred_element_type=jnp.float32)
             + bp[...])
    out_n = (jnp.dot(ap2, wnv[:d], preferred_element_type=jnp.float32)
             + jnp.dot(an1, wnv[d:2 * d], preferred_element_type=jnp.float32)
             + jnp.dot(x2[...], wnv[2 * d:], preferred_element_type=jnp.float32)
             + bn[...])
    hv = h[...]
    o1[...] = (jnp.maximum(out_p, 0.0) + hv) * 0.5
    o2[...] = (jnp.maximum(out_n, 0.0) + hv) * 0.5


def _decq_body(z1, z2, mu, q1, q2):
    muv = mu[...]
    mu2 = jnp.sum(muv * muv, axis=1)[None, :]
    for z, q in ((z1, q1), (z2, q2)):
        zv = z[...]
        z2s = jnp.sum(zv * zv, axis=1, keepdims=True)
        cross = lax.dot_general(zv, muv, (((1,), (1,)), ((), ())),
                                preferred_element_type=jnp.float32)
        d2 = z2s + mu2 - 2.0 * cross
        qv = 1.0 / (1.0 + jnp.maximum(d2, 0.0))
        q[...] = qv / jnp.sum(qv, axis=1, keepdims=True)


def _row_spec(cols):
    return pl.BlockSpec((_TCB, cols), lambda i: (i, 0))


def _full_spec(r, c):
    return pl.BlockSpec((r, c), lambda i: (0, 0))


def _tc_layer1(sp, sn, cp, cn, x, wp, bp, wn, bn, h):
    g = N // _TCB
    dout = 32
    return pl.pallas_call(
        _l1_body,
        grid=(g,),
        in_specs=[
            _row_spec(16), _row_spec(16), _row_spec(1), _row_spec(1),
            _row_spec(3),
            _full_spec(6, dout), _full_spec(1, dout),
            _full_spec(6, dout), _full_spec(1, dout),
            _row_spec(dout),
        ],
        out_specs=[_row_spec(dout), _row_spec(dout)],
        out_shape=[jax.ShapeDtypeStruct((N, dout), jnp.float32)] * 2,
    )(sp, sn, cp, cn, x, wp, bp, wn, bn, h)


def _tc_mid(d, dout, sp1, sp2, sn1, sn2, cp, cn, x1, x2, wp, bp, wn, bn, h):
    g = N // _TCB
    return pl.pallas_call(
        functools.partial(_mid_body, d),
        grid=(g,),
        in_specs=[
            _row_spec(d), _row_spec(d), _row_spec(d), _row_spec(d),
            _row_spec(1), _row_spec(1),
            _row_spec(d), _row_spec(d),
            _full_spec(3 * d, dout), _full_spec(1, dout),
            _full_spec(3 * d, dout), _full_spec(1, dout),
            _row_spec(dout),
        ],
        out_specs=[_row_spec(dout), _row_spec(dout)],
        out_shape=[jax.ShapeDtypeStruct((N, dout), jnp.float32)] * 2,
    )(sp1, sp2, sn1, sn2, cp, cn, x1, x2, wp, bp, wn, bn, h)


def _tc_decq(z1, z2, mu):
    g = N // _TCB
    d = IN_DIMS[4]
    return pl.pallas_call(
        _decq_body,
        grid=(g,),
        in_specs=[
            _row_spec(d), _row_spec(d),
            _full_spec(N_CLUSTERS, d),
        ],
        out_specs=[_row_spec(N_CLUSTERS), _row_spec(N_CLUSTERS)],
        out_shape=[jax.ShapeDtypeStruct((N, N_CLUSTERS), jnp.float32)] * 2,
    )(z1, z2, mu)


def _pad_edges(src, dst):
    pad = EPAD - src.shape[0]
    src_p = jnp.concatenate([src, jnp.zeros((pad,), jnp.int32)])
    dst_p = jnp.concatenate([dst, jnp.full((pad,), N, jnp.int32)])
    return (src_p.reshape(NW, NCHUNK, C), dst_p.reshape(NW, NCHUNK, C))


def kernel(x, edge_index_sim, edge_index_disim, h1, h2, h3, h4,
           W1p, b1p, W1n, b1n, W2p, b2p, W2n, b2n, W3p, b3p, W3n, b3n,
           W4p, b4p, W4n, b4n, mu):
    loops = jnp.arange(N, dtype=jnp.int32)
    sp_s, dp_s = _pad_edges(
        jnp.concatenate([edge_index_sim[0], loops]),
        jnp.concatenate([edge_index_sim[1], loops]))
    sn_s, dn_s = _pad_edges(edge_index_disim[0], edge_index_disim[1])
    src_all = jnp.stack([sp_s, sn_s])
    dst_all = jnp.stack([dp_s, dn_s])

    x16 = jnp.pad(x, ((0, 0), (0, 13)))
    zc16 = jnp.zeros((ROWS_PT, 16), jnp.float32)
    ones16 = jnp.ones((C, 16), jnp.float32)

    o = _make_seg1()(x16, src_all, dst_all, zc16, ones16)
    cp = o[0, 1, :N, 0:1]
    cn = o[1, 1, :N, 0:1]
    x1, x2 = _tc_layer1(o[0, 0, :N], o[1, 0, :N], cp, cn, x,
                        W1p, b1p.reshape(1, -1), W1n, b1n.reshape(1, -1), h1)

    params = [(W2p, b2p, W2n, b2n, h2), (W3p, b3p, W3n, b3n, h3),
              (W4p, b4p, W4n, b4n, h4)]
    for i in range(1, 4):
        d = IN_DIMS[i]
        dout = IN_DIMS[i + 1]
        wp, bp, wn, bn, h = params[i - 1]
        zc = jnp.zeros((ROWS_PT, d), jnp.float32)
        s = _make_seg2(d)(x1, x2, src_all, dst_all, zc)
        x1, x2 = _tc_mid(d, dout, s[0, 0, :N], s[0, 1, :N],
                         s[1, 0, :N], s[1, 1, :N], cp, cn, x1, x2,
                         wp, bp.reshape(1, -1), wn, bn.reshape(1, -1), h)

    q1, q2 = _tc_decq(x1, x2, mu)
    return q1, q2
